# baseline (speedup 1.0000x reference)
import functools

import jax
import jax.numpy as jnp
from jax.experimental import pallas as pl
from jax.experimental.pallas import tpu as pltpu
from jax.experimental.pallas import tpu_sc as plsc

T = 2048
D_IN = 768
D_H = 1536
D_OUT = 768
E = 8
K = 3

B = 256
NBLK = 32
R = NBLK * B
NPAIR = K * T
NW = 32
WIN = R // NW
TW = T // NW
NBUF = 3


def _routing_kernel(x_ref, wg_ref, bg_ref, posk_ref, wk_ref, meta_ref):
    scores = jnp.dot(x_ref[...].astype(jnp.bfloat16),
                     wg_ref[...].astype(jnp.bfloat16),
                     preferred_element_type=jnp.float32) + bg_ref[...]
    lane = jax.lax.broadcasted_iota(jnp.int32, scores.shape, 1)
    neg = jnp.float32(-3.4e38)
    s = scores
    mask = jnp.zeros(scores.shape, dtype=jnp.bool_)
    sels = []
    for _ in range(K):
        m = jnp.max(s, axis=1, keepdims=True)
        is_max = s == m
        first = jnp.min(jnp.where(is_max, lane, E), axis=1, keepdims=True)
        sel = lane == first
        sels.append(sel)
        mask = jnp.logical_or(mask, sel)
        s = jnp.where(sel, neg, s)
    p = jax.nn.softmax(scores, axis=1)
    w = jnp.where(mask, p, 0.0)
    w = w / (jnp.sum(w, axis=1, keepdims=True) + 1e-8)

    maskf = mask.astype(jnp.float32)
    li = jax.lax.broadcasted_iota(jnp.int32, (B, B), 0)
    lj = jax.lax.broadcasted_iota(jnp.int32, (B, B), 1)
    lt_inc = (lj <= li).astype(jnp.bfloat16)
    off = jnp.zeros((1, E), jnp.float32)
    rank_chunks = []
    for c in range(T // B):
        chunk = maskf[c * B:(c + 1) * B, :]
        csum = jnp.dot(lt_inc, chunk.astype(jnp.bfloat16),
                       preferred_element_type=jnp.float32)
        rank_chunks.append(off + csum - chunk)
        off = off + csum[B - 1:B, :]
    rank = jnp.concatenate(rank_chunks, axis=0)
    cnt = off

    nblk = jnp.floor((cnt + (B - 1.0)) / B)
    ei = jax.lax.broadcasted_iota(jnp.int32, (E, E), 0)
    ej = jax.lax.broadcasted_iota(jnp.int32, (E, E), 1)
    lt_strict = (ei < ej).astype(jnp.bfloat16)
    baseblk = jnp.dot(nblk.astype(jnp.bfloat16), lt_strict,
                      preferred_element_type=jnp.float32)
    posf = baseblk * B + rank

    for k in range(K):
        selk = sels[k].astype(jnp.float32)
        pk = jnp.sum(selk * posf, axis=1, keepdims=True)
        wk = jnp.sum(selk * w, axis=1, keepdims=True)
        posk_ref[k:k + 1, :] = jnp.transpose(pk).astype(jnp.int32)
        wk_ref[k:k + 1, :] = jnp.transpose(wk)

    jrow = jax.lax.broadcasted_iota(jnp.int32, (E, NBLK), 1).astype(jnp.float32)
    base_t = jnp.transpose(baseblk)
    started = (jrow >= base_t).astype(jnp.float32)
    active_e = (jnp.transpose(nblk) > 0).astype(jnp.float32)
    blk_e = jnp.sum(started, axis=0, keepdims=True) - 1.0
    runidx = jnp.sum(started * active_e, axis=0, keepdims=True) - 1.0
    slot = runidx - NBUF * jnp.floor(runidx / NBUF)
    total = jnp.sum(nblk, axis=1, keepdims=True)
    meta_ref[0:1, :] = blk_e.astype(jnp.int32)
    meta_ref[1:2, :] = slot.astype(jnp.int32)
    meta_ref[2:3, :] = jnp.broadcast_to(total, (1, NBLK)).astype(jnp.int32)


def _dispatch_kernel(posk_hbm, wk_hbm, x_hbm, xs_out, ws_out,
                     xrows, p0, p1, p2, v0, v1, v2,
                     sa0, sa1, sa2, sb0, sb1, sb2):
    wid = jax.lax.axis_index("s") * 2 + jax.lax.axis_index("c")
    t0 = wid * TW

    pltpu.sync_copy(x_hbm.at[pl.ds(t0, TW), :], xrows)
    pidx = (p0, p1, p2)
    vals = (v0, v1, v2)
    for k in range(K):
        pltpu.sync_copy(posk_hbm.at[pl.ds(k * T + t0, TW)], pidx[k])
        pltpu.sync_copy(wk_hbm.at[pl.ds(k * T + t0, TW)], vals[k])
    copies = []
    for k, (sa, sb) in enumerate(((sa0, sb0), (sa1, sb1), (sa2, sb2))):
        copies.append(pltpu.async_copy(xrows, xs_out.at[pidx[k]], sa))
        copies.append(pltpu.async_copy(vals[k], ws_out.at[pidx[k]], sb))
    for cp in copies:
        cp.wait()


def _w_copy(w1_hbm, w2_hbm, w1r, w2r, sem1, sem2, expert, slot):
    c1 = pltpu.make_async_copy(w1_hbm.at[expert], w1r.at[slot], sem1.at[slot])
    c2 = pltpu.make_async_copy(w2_hbm.at[expert], w2r.at[slot], sem2.at[slot])
    return c1, c2


def _ffn_kernel(be_ref, bs_ref, xs_ref, ws_ref, w1_hbm, b1_ref, w2_hbm,
                b2_ref, out_ref, w1r, w2r, sem1, sem2):
    j = pl.program_id(0)
    cur = be_ref[j]
    slot = bs_ref[j]
    jm1 = jnp.maximum(j - 1, 0)

    @pl.when(j == 0)
    def _():
        c1, c2 = _w_copy(w1_hbm, w2_hbm, w1r, w2r, sem1, sem2,
                         be_ref[0], bs_ref[0])
        c1.start()
        c2.start()
        for kk in range(1, NBUF - 1):
            @pl.when(be_ref[kk] != be_ref[kk - 1])
            def _():
                c1, c2 = _w_copy(w1_hbm, w2_hbm, w1r, w2r, sem1, sem2,
                                 be_ref[kk], bs_ref[kk])
                c1.start()
                c2.start()

    lead = NBUF - 1

    @pl.when(jnp.logical_and(j + lead < NBLK,
                             be_ref[jnp.minimum(j + lead, NBLK - 1)]
                             != be_ref[jnp.minimum(j + lead - 1, NBLK - 1)]))
    def _():
        nj = jnp.minimum(j + lead, NBLK - 1)
        c1, c2 = _w_copy(w1_hbm, w2_hbm, w1r, w2r, sem1, sem2,
                         be_ref[nj], bs_ref[nj])
        c1.start()
        c2.start()

    @pl.when(jnp.logical_or(j == 0, cur != be_ref[jm1]))
    def _():
        c1, c2 = _w_copy(w1_hbm, w2_hbm, w1r, w2r, sem1, sem2, cur, slot)
        c1.wait()
        c2.wait()

    xs = xs_ref[...].astype(jnp.bfloat16)
    h = jnp.dot(xs, w1r[slot].astype(jnp.bfloat16),
                preferred_element_type=jnp.float32)
    h = jnp.maximum(h + b1_ref[0], 0.0)
    o = jnp.dot(h.astype(jnp.bfloat16), w2r[slot].astype(jnp.bfloat16),
                preferred_element_type=jnp.float32)
    o = o + b2_ref[0]
    out_ref[...] = o * ws_ref[0]


def _combine_kernel(os_hbm, posk_hbm, res_out,
                    g0, g1, g2, i0, i1, i2, s0, s1, s2):
    wid = jax.lax.axis_index("s") * 2 + jax.lax.axis_index("c")
    t0 = wid * TW
    gs = (g0, g1, g2)
    idxs = (i0, i1, i2)
    sems = (s0, s1, s2)
    cw = TW // 4
    for c in range(4):
        base = t0 + c * cw
        for k in range(K):
            pltpu.sync_copy(posk_hbm.at[pl.ds(k * T + base, cw)], idxs[k])
        copies = [pltpu.async_copy(os_hbm.at[idxs[k]], gs[k], sems[k])
                  for k in range(K)]
        for cp in copies:
            cp.wait()

        def add_body(i, _):
            r = i // (D_OUT // 16)
            cc = i % (D_OUT // 16)
            sl = pl.ds(cc * 16, 16)
            g0[r, sl] = g0[r, sl] + g1[r, sl] + g2[r, sl]
            return 0

        jax.lax.fori_loop(0, cw * D_OUT // 16, add_body, 0)
        pltpu.sync_copy(g0, res_out.at[pl.ds(base, cw), :])


def kernel(x, Wg, bg, W1, b1, W2, b2):
    bg2 = bg.reshape(1, E)
    posk, wk, meta = pl.pallas_call(
        _routing_kernel,
        out_shape=(
            jax.ShapeDtypeStruct((K, T), jnp.int32),
            jax.ShapeDtypeStruct((K, T), jnp.float32),
            jax.ShapeDtypeStruct((K, NBLK), jnp.int32),
        ),
        in_specs=[
            pl.BlockSpec((T, D_IN), lambda: (0, 0)),
            pl.BlockSpec((D_IN, E), lambda: (0, 0)),
            pl.BlockSpec((1, E), lambda: (0, 0)),
        ],
        out_specs=(
            pl.BlockSpec((K, T), lambda: (0, 0)),
            pl.BlockSpec((K, T), lambda: (0, 0)),
            pl.BlockSpec((K, NBLK), lambda: (0, 0)),
        ),
    )(x, Wg, bg2)

    mesh = plsc.VectorSubcoreMesh(core_axis_name="c", subcore_axis_name="s")
    dispatch = functools.partial(
        pl.kernel,
        out_type=(
            jax.ShapeDtypeStruct((R, D_IN), jnp.float32),
            jax.ShapeDtypeStruct((R,), jnp.float32),
        ),
        mesh=mesh,
        scratch_types=[
            pltpu.VMEM((TW, D_IN), jnp.float32),
            pltpu.VMEM((TW,), jnp.int32),
            pltpu.VMEM((TW,), jnp.int32),
            pltpu.VMEM((TW,), jnp.int32),
            pltpu.VMEM((TW,), jnp.float32),
            pltpu.VMEM((TW,), jnp.float32),
            pltpu.VMEM((TW,), jnp.float32),
            pltpu.SemaphoreType.DMA,
            pltpu.SemaphoreType.DMA,
            pltpu.SemaphoreType.DMA,
            pltpu.SemaphoreType.DMA,
            pltpu.SemaphoreType.DMA,
            pltpu.SemaphoreType.DMA,
        ],
    )(_dispatch_kernel)
    x_sorted, w_sorted = dispatch(posk.reshape(NPAIR), wk.reshape(NPAIR), x)

    be = meta[0]
    bs = meta[1]
    out_sorted = pl.pallas_call(
        _ffn_kernel,
        grid_spec=pltpu.PrefetchScalarGridSpec(
            num_scalar_prefetch=2,
            grid=(NBLK,),
            in_specs=[
                pl.BlockSpec((B, D_IN), lambda j, be, bs: (j, 0)),
                pl.BlockSpec((1, B, 1), lambda j, be, bs: (j, 0, 0)),
                pl.BlockSpec(memory_space=pl.ANY),
                pl.BlockSpec((1, 1, D_H), lambda j, be, bs: (be[j], 0, 0)),
                pl.BlockSpec(memory_space=pl.ANY),
                pl.BlockSpec((1, 1, D_OUT), lambda j, be, bs: (be[j], 0, 0)),
            ],
            out_specs=pl.BlockSpec((B, D_OUT), lambda j, be, bs: (j, 0)),
            scratch_shapes=[
                pltpu.VMEM((NBUF, D_IN, D_H), jnp.float32),
                pltpu.VMEM((NBUF, D_H, D_OUT), jnp.float32),
                pltpu.SemaphoreType.DMA((NBUF,)),
                pltpu.SemaphoreType.DMA((NBUF,)),
            ],
        ),
        out_shape=jax.ShapeDtypeStruct((R, D_OUT), jnp.float32),
        compiler_params=pltpu.CompilerParams(
            dimension_semantics=("arbitrary",),
        ),
    )(be, bs, x_sorted, w_sorted.reshape(NBLK, B, 1),
      W1, b1.reshape(E, 1, D_H), W2, b2.reshape(E, 1, D_OUT))

    combine = functools.partial(
        pl.kernel,
        out_type=jax.ShapeDtypeStruct((T, D_OUT), jnp.float32),
        mesh=mesh,
        scratch_types=[
            pltpu.VMEM((TW // 4, D_OUT), jnp.float32),
            pltpu.VMEM((TW // 4, D_OUT), jnp.float32),
            pltpu.VMEM((TW // 4, D_OUT), jnp.float32),
            pltpu.VMEM((TW // 4,), jnp.int32),
            pltpu.VMEM((TW // 4,), jnp.int32),
            pltpu.VMEM((TW // 4,), jnp.int32),
            pltpu.SemaphoreType.DMA,
            pltpu.SemaphoreType.DMA,
            pltpu.SemaphoreType.DMA,
        ],
    )(_combine_kernel)
    result = combine(out_sorted, posk.reshape(NPAIR))
    return result


# speedup vs baseline: 1.2751x; 1.2751x over previous
import functools

import jax
import jax.numpy as jnp
from jax.experimental import pallas as pl
from jax.experimental.pallas import tpu as pltpu
from jax.experimental.pallas import tpu_sc as plsc

T = 2048
D_IN = 768
D_H = 1536
D_OUT = 768
E = 8
K = 3

B = 256
NBLK = 32
R = NBLK * B
NPAIR = K * T
NW = 32
WIN = R // NW
TW = T // NW
NBUF = 3


def _routing_kernel(x_ref, wg_ref, bg_ref, posk_ref, wk_ref, meta_ref,
                    wbig_ref):
    scores = jnp.dot(x_ref[...].astype(jnp.bfloat16),
                     wg_ref[...].astype(jnp.bfloat16),
                     preferred_element_type=jnp.float32) + bg_ref[...]
    lane = jax.lax.broadcasted_iota(jnp.int32, scores.shape, 1)
    neg = jnp.float32(-3.4e38)
    s = scores
    mask = jnp.zeros(scores.shape, dtype=jnp.bool_)
    sels = []
    for _ in range(K):
        m = jnp.max(s, axis=1, keepdims=True)
        is_max = s == m
        first = jnp.min(jnp.where(is_max, lane, E), axis=1, keepdims=True)
        sel = lane == first
        sels.append(sel)
        mask = jnp.logical_or(mask, sel)
        s = jnp.where(sel, neg, s)
    p = jax.nn.softmax(scores, axis=1)
    w = jnp.where(mask, p, 0.0)
    w = w / (jnp.sum(w, axis=1, keepdims=True) + 1e-8)

    maskf = mask.astype(jnp.float32)
    li = jax.lax.broadcasted_iota(jnp.int32, (B, B), 0)
    lj = jax.lax.broadcasted_iota(jnp.int32, (B, B), 1)
    lt_inc = (lj <= li).astype(jnp.bfloat16)
    off = jnp.zeros((1, E), jnp.float32)
    rank_chunks = []
    for c in range(T // B):
        chunk = maskf[c * B:(c + 1) * B, :]
        csum = jnp.dot(lt_inc, chunk.astype(jnp.bfloat16),
                       preferred_element_type=jnp.float32)
        rank_chunks.append(off + csum - chunk)
        off = off + csum[B - 1:B, :]
    rank = jnp.concatenate(rank_chunks, axis=0)
    cnt = off

    nblk = jnp.floor((cnt + (B - 1.0)) / B)
    ei = jax.lax.broadcasted_iota(jnp.int32, (E, E), 0)
    ej = jax.lax.broadcasted_iota(jnp.int32, (E, E), 1)
    lt_strict = (ei < ej).astype(jnp.bfloat16)
    baseblk = jnp.dot(nblk.astype(jnp.bfloat16), lt_strict,
                      preferred_element_type=jnp.float32)
    posf = baseblk * B + rank

    for k in range(K):
        selk = sels[k].astype(jnp.float32)
        pk = jnp.sum(selk * posf, axis=1, keepdims=True)
        wk = jnp.sum(selk * w, axis=1, keepdims=True)
        posk_ref[k:k + 1, :] = jnp.transpose(pk).astype(jnp.int32)
        wk_ref[k:k + 1, :] = jnp.transpose(wk)
        wbig_ref[k * T:(k + 1) * T, :] = jnp.broadcast_to(wk, (T, 16))

    jrow = jax.lax.broadcasted_iota(jnp.int32, (E, NBLK), 1).astype(jnp.float32)
    base_t = jnp.transpose(baseblk)
    started = (jrow >= base_t).astype(jnp.float32)
    active_e = (jnp.transpose(nblk) > 0).astype(jnp.float32)
    blk_e = jnp.sum(started, axis=0, keepdims=True) - 1.0
    runidx = jnp.sum(started * active_e, axis=0, keepdims=True) - 1.0
    slot = runidx - NBUF * jnp.floor(runidx / NBUF)
    total = jnp.sum(nblk, axis=1, keepdims=True)
    meta_ref[0:1, :] = blk_e.astype(jnp.int32)
    meta_ref[1:2, :] = slot.astype(jnp.int32)
    meta_ref[2:3, :] = jnp.broadcast_to(total, (1, NBLK)).astype(jnp.int32)


def _dispatch_kernel(posk_hbm, x_hbm, xs_out,
                     xrows, p0, p1, p2, sa0, sa1, sa2):
    wid = jax.lax.axis_index("s") * 2 + jax.lax.axis_index("c")
    t0 = wid * TW

    pltpu.sync_copy(x_hbm.at[pl.ds(t0, TW), :], xrows)
    pidx = (p0, p1, p2)
    for k in range(K):
        pltpu.sync_copy(posk_hbm.at[pl.ds(k * T + t0, TW)], pidx[k])
    copies = []
    for k, sa in enumerate((sa0, sa1, sa2)):
        copies.append(pltpu.async_copy(xrows, xs_out.at[pidx[k]], sa))
    for cp in copies:
        cp.wait()


def _w_copy(w1_hbm, w2_hbm, w1r, w2r, sem1, sem2, expert, slot):
    c1 = pltpu.make_async_copy(w1_hbm.at[expert], w1r.at[slot], sem1.at[slot])
    c2 = pltpu.make_async_copy(w2_hbm.at[expert], w2r.at[slot], sem2.at[slot])
    return c1, c2


def _ffn_kernel(be_ref, bs_ref, xs_ref, w1_hbm, b1_ref, w2_hbm,
                b2_ref, out_ref, w1r, w2r, sem1, sem2):
    j = pl.program_id(0)
    cur = be_ref[j]
    slot = bs_ref[j]
    jm1 = jnp.maximum(j - 1, 0)

    @pl.when(j == 0)
    def _():
        c1, c2 = _w_copy(w1_hbm, w2_hbm, w1r, w2r, sem1, sem2,
                         be_ref[0], bs_ref[0])
        c1.start()
        c2.start()
        for kk in range(1, NBUF - 1):
            @pl.when(be_ref[kk] != be_ref[kk - 1])
            def _():
                c1, c2 = _w_copy(w1_hbm, w2_hbm, w1r, w2r, sem1, sem2,
                                 be_ref[kk], bs_ref[kk])
                c1.start()
                c2.start()

    lead = NBUF - 1

    @pl.when(jnp.logical_and(j + lead < NBLK,
                             be_ref[jnp.minimum(j + lead, NBLK - 1)]
                             != be_ref[jnp.minimum(j + lead - 1, NBLK - 1)]))
    def _():
        nj = jnp.minimum(j + lead, NBLK - 1)
        c1, c2 = _w_copy(w1_hbm, w2_hbm, w1r, w2r, sem1, sem2,
                         be_ref[nj], bs_ref[nj])
        c1.start()
        c2.start()

    @pl.when(jnp.logical_or(j == 0, cur != be_ref[jm1]))
    def _():
        c1, c2 = _w_copy(w1_hbm, w2_hbm, w1r, w2r, sem1, sem2, cur, slot)
        c1.wait()
        c2.wait()

    xs = xs_ref[...].astype(jnp.bfloat16)
    h = jnp.dot(xs, w1r[slot].astype(jnp.bfloat16),
                preferred_element_type=jnp.float32)
    h = jnp.maximum(h + b1_ref[0], 0.0)
    o = jnp.dot(h.astype(jnp.bfloat16), w2r[slot].astype(jnp.bfloat16),
                preferred_element_type=jnp.float32)
    out_ref[...] = o + b2_ref[0]


def _combine_kernel(os_hbm, posk_hbm, wbig_hbm, res_out,
                    g0, g1, g2, i0, i1, i2, wb0, wb1, wb2, s0, s1, s2):
    wid = jax.lax.axis_index("s") * 2 + jax.lax.axis_index("c")
    t0 = wid * TW
    gs = (g0, g1, g2)
    idxs = (i0, i1, i2)
    wbs = (wb0, wb1, wb2)
    sems = (s0, s1, s2)
    cw = TW // 4
    NC = D_OUT // 16
    for c in range(4):
        base = t0 + c * cw
        for k in range(K):
            pltpu.sync_copy(posk_hbm.at[pl.ds(k * T + base, cw)], idxs[k])
            pltpu.sync_copy(wbig_hbm.at[pl.ds((k * T + base) * 16, cw * 16)],
                            wbs[k])
        copies = [pltpu.async_copy(os_hbm.at[idxs[k]], gs[k], sems[k])
                  for k in range(K)]
        for cp in copies:
            cp.wait()

        def row_body(r, _):
            w0 = wb0[pl.ds(r * 16, 16)]
            w1 = wb1[pl.ds(r * 16, 16)]
            w2 = wb2[pl.ds(r * 16, 16)]
            for c2 in range(NC):
                sl = pl.ds(c2 * 16, 16)
                g0[r, sl] = g0[r, sl] * w0 + g1[r, sl] * w1 + g2[r, sl] * w2
            return 0

        jax.lax.fori_loop(0, cw, row_body, 0)
        pltpu.sync_copy(g0, res_out.at[pl.ds(base, cw), :])


def kernel(x, Wg, bg, W1, b1, W2, b2):
    bg2 = bg.reshape(1, E)
    posk, wk, meta, wbig = pl.pallas_call(
        _routing_kernel,
        out_shape=(
            jax.ShapeDtypeStruct((K, T), jnp.int32),
            jax.ShapeDtypeStruct((K, T), jnp.float32),
            jax.ShapeDtypeStruct((K, NBLK), jnp.int32),
            jax.ShapeDtypeStruct((K * T, 16), jnp.float32),
        ),
        in_specs=[
            pl.BlockSpec((T, D_IN), lambda: (0, 0)),
            pl.BlockSpec((D_IN, E), lambda: (0, 0)),
            pl.BlockSpec((1, E), lambda: (0, 0)),
        ],
        out_specs=(
            pl.BlockSpec((K, T), lambda: (0, 0)),
            pl.BlockSpec((K, T), lambda: (0, 0)),
            pl.BlockSpec((K, NBLK), lambda: (0, 0)),
            pl.BlockSpec((K * T, 16), lambda: (0, 0)),
        ),
    )(x, Wg, bg2)

    mesh = plsc.VectorSubcoreMesh(core_axis_name="c", subcore_axis_name="s")
    dispatch = functools.partial(
        pl.kernel,
        out_type=jax.ShapeDtypeStruct((R, D_IN), jnp.float32),
        mesh=mesh,
        scratch_types=[
            pltpu.VMEM((TW, D_IN), jnp.float32),
            pltpu.VMEM((TW,), jnp.int32),
            pltpu.VMEM((TW,), jnp.int32),
            pltpu.VMEM((TW,), jnp.int32),
            pltpu.SemaphoreType.DMA,
            pltpu.SemaphoreType.DMA,
            pltpu.SemaphoreType.DMA,
        ],
    )(_dispatch_kernel)
    x_sorted = dispatch(posk.reshape(NPAIR), x)

    be = meta[0]
    bs = meta[1]
    out_sorted = pl.pallas_call(
        _ffn_kernel,
        grid_spec=pltpu.PrefetchScalarGridSpec(
            num_scalar_prefetch=2,
            grid=(NBLK,),
            in_specs=[
                pl.BlockSpec((B, D_IN), lambda j, be, bs: (j, 0)),
                pl.BlockSpec(memory_space=pl.ANY),
                pl.BlockSpec((1, 1, D_H), lambda j, be, bs: (be[j], 0, 0)),
                pl.BlockSpec(memory_space=pl.ANY),
                pl.BlockSpec((1, 1, D_OUT), lambda j, be, bs: (be[j], 0, 0)),
            ],
            out_specs=pl.BlockSpec((B, D_OUT), lambda j, be, bs: (j, 0)),
            scratch_shapes=[
                pltpu.VMEM((NBUF, D_IN, D_H), jnp.float32),
                pltpu.VMEM((NBUF, D_H, D_OUT), jnp.float32),
                pltpu.SemaphoreType.DMA((NBUF,)),
                pltpu.SemaphoreType.DMA((NBUF,)),
            ],
        ),
        out_shape=jax.ShapeDtypeStruct((R, D_OUT), jnp.float32),
        compiler_params=pltpu.CompilerParams(
            dimension_semantics=("arbitrary",),
        ),
    )(be, bs, x_sorted,
      W1, b1.reshape(E, 1, D_H), W2, b2.reshape(E, 1, D_OUT))

    combine = functools.partial(
        pl.kernel,
        out_type=jax.ShapeDtypeStruct((T, D_OUT), jnp.float32),
        mesh=mesh,
        scratch_types=[
            pltpu.VMEM((TW // 4, D_OUT), jnp.float32),
            pltpu.VMEM((TW // 4, D_OUT), jnp.float32),
            pltpu.VMEM((TW // 4, D_OUT), jnp.float32),
            pltpu.VMEM((TW // 4,), jnp.int32),
            pltpu.VMEM((TW // 4,), jnp.int32),
            pltpu.VMEM((TW // 4,), jnp.int32),
            pltpu.VMEM((TW * 4,), jnp.float32),
            pltpu.VMEM((TW * 4,), jnp.float32),
            pltpu.VMEM((TW * 4,), jnp.float32),
            pltpu.SemaphoreType.DMA,
            pltpu.SemaphoreType.DMA,
            pltpu.SemaphoreType.DMA,
        ],
    )(_combine_kernel)
    result = combine(out_sorted, posk.reshape(NPAIR), wbig.reshape(K * T * 16))
    return result


# speedup vs baseline: 1.3543x; 1.0621x over previous
import functools

import jax
import jax.numpy as jnp
from jax.experimental import pallas as pl
from jax.experimental.pallas import tpu as pltpu
from jax.experimental.pallas import tpu_sc as plsc

T = 2048
D_IN = 768
D_H = 1536
D_OUT = 768
E = 8
K = 3

B = 256
NBLK = 32
R = NBLK * B
NPAIR = K * T
NW = 32
WIN = R // NW
TW = T // NW
NBUF = 3


def _routing_kernel(x_ref, wg_ref, bg_ref, posk_ref, wk_ref, meta_ref,
                    wbig_ref):
    scores = jnp.dot(x_ref[...].astype(jnp.bfloat16),
                     wg_ref[...].astype(jnp.bfloat16),
                     preferred_element_type=jnp.float32) + bg_ref[...]
    lane = jax.lax.broadcasted_iota(jnp.int32, scores.shape, 1)
    neg = jnp.float32(-3.4e38)
    s = scores
    mask = jnp.zeros(scores.shape, dtype=jnp.bool_)
    sels = []
    for _ in range(K):
        m = jnp.max(s, axis=1, keepdims=True)
        is_max = s == m
        first = jnp.min(jnp.where(is_max, lane, E), axis=1, keepdims=True)
        sel = lane == first
        sels.append(sel)
        mask = jnp.logical_or(mask, sel)
        s = jnp.where(sel, neg, s)
    p = jax.nn.softmax(scores, axis=1)
    w = jnp.where(mask, p, 0.0)
    w = w / (jnp.sum(w, axis=1, keepdims=True) + 1e-8)

    maskf = mask.astype(jnp.float32)
    li = jax.lax.broadcasted_iota(jnp.int32, (B, B), 0)
    lj = jax.lax.broadcasted_iota(jnp.int32, (B, B), 1)
    lt_inc = (lj <= li).astype(jnp.bfloat16)
    off = jnp.zeros((1, E), jnp.float32)
    rank_chunks = []
    for c in range(T // B):
        chunk = maskf[c * B:(c + 1) * B, :]
        csum = jnp.dot(lt_inc, chunk.astype(jnp.bfloat16),
                       preferred_element_type=jnp.float32)
        rank_chunks.append(off + csum - chunk)
        off = off + csum[B - 1:B, :]
    rank = jnp.concatenate(rank_chunks, axis=0)
    cnt = off

    nblk = jnp.floor((cnt + (B - 1.0)) / B)
    ei = jax.lax.broadcasted_iota(jnp.int32, (E, E), 0)
    ej = jax.lax.broadcasted_iota(jnp.int32, (E, E), 1)
    lt_strict = (ei < ej).astype(jnp.bfloat16)
    baseblk = jnp.dot(nblk.astype(jnp.bfloat16), lt_strict,
                      preferred_element_type=jnp.float32)
    posf = baseblk * B + rank

    for k in range(K):
        selk = sels[k].astype(jnp.float32)
        pk = jnp.sum(selk * posf, axis=1, keepdims=True)
        wk = jnp.sum(selk * w, axis=1, keepdims=True)
        posk_ref[k:k + 1, :] = jnp.transpose(pk).astype(jnp.int32)
        wk_ref[k:k + 1, :] = jnp.transpose(wk)
        wbig_ref[k * T:(k + 1) * T, :] = jnp.broadcast_to(wk, (T, 16))

    jrow = jax.lax.broadcasted_iota(jnp.int32, (E, NBLK), 1).astype(jnp.float32)
    base_t = jnp.transpose(baseblk)
    started = (jrow >= base_t).astype(jnp.float32)
    active_e = (jnp.transpose(nblk) > 0).astype(jnp.float32)
    blk_e = jnp.sum(started, axis=0, keepdims=True) - 1.0
    runidx = jnp.sum(started * active_e, axis=0, keepdims=True) - 1.0
    slot = runidx - NBUF * jnp.floor(runidx / NBUF)
    total = jnp.sum(nblk, axis=1, keepdims=True)
    meta_ref[0:1, :] = blk_e.astype(jnp.int32)
    meta_ref[1:2, :] = slot.astype(jnp.int32)
    meta_ref[2:3, :] = jnp.broadcast_to(total, (1, NBLK)).astype(jnp.int32)


def _dispatch_kernel(posk_hbm, x_hbm, xs_out,
                     xrows, p0, p1, p2, sa0, sa1, sa2):
    wid = jax.lax.axis_index("s") * 2 + jax.lax.axis_index("c")
    t0 = wid * TW

    pltpu.sync_copy(x_hbm.at[pl.ds(t0, TW), :], xrows)
    pidx = (p0, p1, p2)
    for k in range(K):
        pltpu.sync_copy(posk_hbm.at[pl.ds(k * T + t0, TW)], pidx[k])
    copies = []
    for k, sa in enumerate((sa0, sa1, sa2)):
        copies.append(pltpu.async_copy(xrows, xs_out.at[pidx[k]], sa))
    for cp in copies:
        cp.wait()


def _w_copy(w1_hbm, w2_hbm, w1r, w2r, sem1, sem2, expert, slot):
    c1 = pltpu.make_async_copy(w1_hbm.at[expert], w1r.at[slot], sem1.at[slot])
    c2 = pltpu.make_async_copy(w2_hbm.at[expert], w2r.at[slot], sem2.at[slot])
    return c1, c2


def _ffn_kernel(be_ref, bs_ref, nt_ref, xs_ref, w1_hbm, b1_ref, w2_hbm,
                b2_ref, out_ref, w1r, w2r, sem1, sem2):
    j = pl.program_id(0)
    cur = be_ref[j]
    slot = bs_ref[j]
    jm1 = jnp.maximum(j - 1, 0)

    @pl.when(j == 0)
    def _():
        c1, c2 = _w_copy(w1_hbm, w2_hbm, w1r, w2r, sem1, sem2,
                         be_ref[0], bs_ref[0])
        c1.start()
        c2.start()
        for kk in range(1, NBUF - 1):
            @pl.when(be_ref[kk] != be_ref[kk - 1])
            def _():
                c1, c2 = _w_copy(w1_hbm, w2_hbm, w1r, w2r, sem1, sem2,
                                 be_ref[kk], bs_ref[kk])
                c1.start()
                c2.start()

    lead = NBUF - 1

    @pl.when(jnp.logical_and(j + lead < NBLK,
                             be_ref[jnp.minimum(j + lead, NBLK - 1)]
                             != be_ref[jnp.minimum(j + lead - 1, NBLK - 1)]))
    def _():
        nj = jnp.minimum(j + lead, NBLK - 1)
        c1, c2 = _w_copy(w1_hbm, w2_hbm, w1r, w2r, sem1, sem2,
                         be_ref[nj], bs_ref[nj])
        c1.start()
        c2.start()

    @pl.when(jnp.logical_or(j == 0, cur != be_ref[jm1]))
    def _():
        c1, c2 = _w_copy(w1_hbm, w2_hbm, w1r, w2r, sem1, sem2, cur, slot)
        c1.wait()
        c2.wait()

    @pl.when(j < nt_ref[0])
    def _():
        xs = xs_ref[...].astype(jnp.bfloat16)
        h = jnp.dot(xs, w1r[slot].astype(jnp.bfloat16),
                    preferred_element_type=jnp.float32)
        h = jnp.maximum(h + b1_ref[0], 0.0)
        o = jnp.dot(h.astype(jnp.bfloat16), w2r[slot].astype(jnp.bfloat16),
                    preferred_element_type=jnp.float32)
        out_ref[...] = o + b2_ref[0]


def _combine_kernel(os_hbm, posk_hbm, wbig_hbm, res_out,
                    g0a, g1a, g2a, g0b, g1b, g2b,
                    i0a, i1a, i2a, i0b, i1b, i2b,
                    wb0, wb1, wb2,
                    s0a, s1a, s2a, s0b, s1b, s2b):
    wid = jax.lax.axis_index("s") * 2 + jax.lax.axis_index("c")
    t0 = wid * TW
    gsets = ((g0a, g1a, g2a), (g0b, g1b, g2b))
    isets = ((i0a, i1a, i2a), (i0b, i1b, i2b))
    ssets = ((s0a, s1a, s2a), (s0b, s1b, s2b))
    wbs = (wb0, wb1, wb2)
    cw = TW // 4
    NC = D_OUT // 16

    def fire(c, par):
        base = t0 + c * cw
        cps = []
        for k in range(K):
            pltpu.sync_copy(posk_hbm.at[pl.ds(k * T + base, cw)],
                            isets[par][k])
            cps.append(pltpu.async_copy(os_hbm.at[isets[par][k]],
                                        gsets[par][k], ssets[par][k]))
        return cps

    inflight = fire(0, 0)
    for c in range(4):
        par = c % 2
        base = t0 + c * cw
        for k in range(K):
            pltpu.sync_copy(wbig_hbm.at[pl.ds((k * T + base) * 16, cw * 16)],
                            wbs[k])
        for cp in inflight:
            cp.wait()
        if c + 1 < 4:
            inflight = fire(c + 1, 1 - par)
        g0, g1, g2 = gsets[par]

        def row_body(r, _):
            w0 = wb0[pl.ds(r * 16, 16)]
            w1 = wb1[pl.ds(r * 16, 16)]
            w2 = wb2[pl.ds(r * 16, 16)]
            for c2 in range(NC):
                sl = pl.ds(c2 * 16, 16)
                g0[r, sl] = g0[r, sl] * w0 + g1[r, sl] * w1 + g2[r, sl] * w2
            return 0

        jax.lax.fori_loop(0, cw, row_body, 0)
        pltpu.sync_copy(g0, res_out.at[pl.ds(base, cw), :])


def kernel(x, Wg, bg, W1, b1, W2, b2):
    bg2 = bg.reshape(1, E)
    posk, wk, meta, wbig = pl.pallas_call(
        _routing_kernel,
        out_shape=(
            jax.ShapeDtypeStruct((K, T), jnp.int32),
            jax.ShapeDtypeStruct((K, T), jnp.float32),
            jax.ShapeDtypeStruct((K, NBLK), jnp.int32),
            jax.ShapeDtypeStruct((K * T, 16), jnp.float32),
        ),
        in_specs=[
            pl.BlockSpec((T, D_IN), lambda: (0, 0)),
            pl.BlockSpec((D_IN, E), lambda: (0, 0)),
            pl.BlockSpec((1, E), lambda: (0, 0)),
        ],
        out_specs=(
            pl.BlockSpec((K, T), lambda: (0, 0)),
            pl.BlockSpec((K, T), lambda: (0, 0)),
            pl.BlockSpec((K, NBLK), lambda: (0, 0)),
            pl.BlockSpec((K * T, 16), lambda: (0, 0)),
        ),
    )(x, Wg, bg2)

    mesh = plsc.VectorSubcoreMesh(core_axis_name="c", subcore_axis_name="s")
    dispatch = functools.partial(
        pl.kernel,
        out_type=jax.ShapeDtypeStruct((R, D_IN), jnp.float32),
        mesh=mesh,
        scratch_types=[
            pltpu.VMEM((TW, D_IN), jnp.float32),
            pltpu.VMEM((TW,), jnp.int32),
            pltpu.VMEM((TW,), jnp.int32),
            pltpu.VMEM((TW,), jnp.int32),
            pltpu.SemaphoreType.DMA,
            pltpu.SemaphoreType.DMA,
            pltpu.SemaphoreType.DMA,
        ],
    )(_dispatch_kernel)
    x_sorted = dispatch(posk.reshape(NPAIR), x)

    be = meta[0]
    bs = meta[1]
    out_sorted = pl.pallas_call(
        _ffn_kernel,
        grid_spec=pltpu.PrefetchScalarGridSpec(
            num_scalar_prefetch=3,
            grid=(NBLK,),
            in_specs=[
                pl.BlockSpec((B, D_IN), lambda j, be, bs, nt: (j, 0)),
                pl.BlockSpec(memory_space=pl.ANY),
                pl.BlockSpec((1, 1, D_H), lambda j, be, bs, nt: (be[j], 0, 0)),
                pl.BlockSpec(memory_space=pl.ANY),
                pl.BlockSpec((1, 1, D_OUT),
                             lambda j, be, bs, nt: (be[j], 0, 0)),
            ],
            out_specs=pl.BlockSpec((B, D_OUT), lambda j, be, bs, nt: (j, 0)),
            scratch_shapes=[
                pltpu.VMEM((NBUF, D_IN, D_H), jnp.float32),
                pltpu.VMEM((NBUF, D_H, D_OUT), jnp.float32),
                pltpu.SemaphoreType.DMA((NBUF,)),
                pltpu.SemaphoreType.DMA((NBUF,)),
            ],
        ),
        out_shape=jax.ShapeDtypeStruct((R, D_OUT), jnp.float32),
        compiler_params=pltpu.CompilerParams(
            dimension_semantics=("arbitrary",),
        ),
    )(be, bs, meta[2, :1], x_sorted,
      W1, b1.reshape(E, 1, D_H), W2, b2.reshape(E, 1, D_OUT))

    combine = functools.partial(
        pl.kernel,
        out_type=jax.ShapeDtypeStruct((T, D_OUT), jnp.float32),
        mesh=mesh,
        scratch_types=(
            [pltpu.VMEM((TW // 4, D_OUT), jnp.float32)] * 6
            + [pltpu.VMEM((TW // 4,), jnp.int32)] * 6
            + [pltpu.VMEM((TW * 4,), jnp.float32)] * 3
            + [pltpu.SemaphoreType.DMA] * 6
        ),
    )(_combine_kernel)
    result = combine(out_sorted, posk.reshape(NPAIR), wbig.reshape(K * T * 16))
    return result


# speedup vs baseline: 1.3641x; 1.0072x over previous
import functools

import jax
import jax.numpy as jnp
from jax.experimental import pallas as pl
from jax.experimental.pallas import tpu as pltpu
from jax.experimental.pallas import tpu_sc as plsc

T = 2048
D_IN = 768
D_H = 1536
D_OUT = 768
E = 8
K = 3

B = 256
NBLK = 32
R = NBLK * B
NPAIR = K * T
NW = 32
WIN = R // NW
TW = T // NW
NBUF = 3


def _routing_kernel(x_ref, wg_ref, bg_ref, posk_ref, wk_ref, meta_ref,
                    wbig_ref):
    scores = jnp.dot(x_ref[...].astype(jnp.bfloat16),
                     wg_ref[...].astype(jnp.bfloat16),
                     preferred_element_type=jnp.float32) + bg_ref[...]
    lane = jax.lax.broadcasted_iota(jnp.int32, scores.shape, 1)
    neg = jnp.float32(-3.4e38)
    s = scores
    mask = jnp.zeros(scores.shape, dtype=jnp.bool_)
    sels = []
    for _ in range(K):
        m = jnp.max(s, axis=1, keepdims=True)
        is_max = s == m
        first = jnp.min(jnp.where(is_max, lane, E), axis=1, keepdims=True)
        sel = lane == first
        sels.append(sel)
        mask = jnp.logical_or(mask, sel)
        s = jnp.where(sel, neg, s)
    p = jax.nn.softmax(scores, axis=1)
    w = jnp.where(mask, p, 0.0)
    w = w / (jnp.sum(w, axis=1, keepdims=True) + 1e-8)

    maskf = mask.astype(jnp.float32)
    li = jax.lax.broadcasted_iota(jnp.int32, (B, B), 0)
    lj = jax.lax.broadcasted_iota(jnp.int32, (B, B), 1)
    lt_inc = (lj <= li).astype(jnp.bfloat16)
    off = jnp.zeros((1, E), jnp.float32)
    rank_chunks = []
    for c in range(T // B):
        chunk = maskf[c * B:(c + 1) * B, :]
        csum = jnp.dot(lt_inc, chunk.astype(jnp.bfloat16),
                       preferred_element_type=jnp.float32)
        rank_chunks.append(off + csum - chunk)
        off = off + csum[B - 1:B, :]
    rank = jnp.concatenate(rank_chunks, axis=0)
    cnt = off

    nblk = jnp.floor((cnt + (B - 1.0)) / B)
    ei = jax.lax.broadcasted_iota(jnp.int32, (E, E), 0)
    ej = jax.lax.broadcasted_iota(jnp.int32, (E, E), 1)
    lt_strict = (ei < ej).astype(jnp.bfloat16)
    baseblk = jnp.dot(nblk.astype(jnp.bfloat16), lt_strict,
                      preferred_element_type=jnp.float32)
    posf = baseblk * B + rank

    for k in range(K):
        selk = sels[k].astype(jnp.float32)
        pk = jnp.sum(selk * posf, axis=1, keepdims=True)
        wk = jnp.sum(selk * w, axis=1, keepdims=True)
        posk_ref[k:k + 1, :] = jnp.transpose(pk).astype(jnp.int32)
        wk_ref[k:k + 1, :] = jnp.transpose(wk)
        wbig_ref[k * T:(k + 1) * T, :] = jnp.broadcast_to(wk, (T, 16))

    jrow = jax.lax.broadcasted_iota(jnp.int32, (E, NBLK), 1).astype(jnp.float32)
    base_t = jnp.transpose(baseblk)
    started = (jrow >= base_t).astype(jnp.float32)
    active_e = (jnp.transpose(nblk) > 0).astype(jnp.float32)
    blk_e = jnp.sum(started, axis=0, keepdims=True) - 1.0
    runidx = jnp.sum(started * active_e, axis=0, keepdims=True) - 1.0
    slot = runidx - NBUF * jnp.floor(runidx / NBUF)
    total = jnp.sum(nblk, axis=1, keepdims=True)
    cand = jnp.where(jnp.logical_and(base_t > jrow, active_e > 0),
                     base_t * jnp.ones((1, NBLK), jnp.float32),
                     jnp.float32(NBLK))
    rs = jnp.min(cand, axis=0, keepdims=True)
    meta_ref[0:1, :] = blk_e.astype(jnp.int32)
    meta_ref[1:2, :] = slot.astype(jnp.int32)
    meta_ref[2:3, :] = jnp.broadcast_to(total, (1, NBLK)).astype(jnp.int32)
    meta_ref[3:4, :] = rs.astype(jnp.int32)


def _dispatch_kernel(posk_hbm, x_hbm, xs_out,
                     xrows, p0, p1, p2, sa0, sa1, sa2):
    wid = jax.lax.axis_index("s") * 2 + jax.lax.axis_index("c")
    t0 = wid * TW

    pltpu.sync_copy(x_hbm.at[pl.ds(t0, TW), :], xrows)
    pidx = (p0, p1, p2)
    for k in range(K):
        pltpu.sync_copy(posk_hbm.at[pl.ds(k * T + t0, TW)], pidx[k])
    copies = []
    for k, sa in enumerate((sa0, sa1, sa2)):
        copies.append(pltpu.async_copy(xrows, xs_out.at[pidx[k]], sa))
    for cp in copies:
        cp.wait()


def _w_copy(w1_hbm, w2_hbm, w1r, w2r, sem1, sem2, expert, slot):
    c1 = pltpu.make_async_copy(w1_hbm.at[expert], w1r.at[slot], sem1.at[slot])
    c2 = pltpu.make_async_copy(w2_hbm.at[expert], w2r.at[slot], sem2.at[slot])
    return c1, c2


def _ffn_kernel(be_ref, bs_ref, nt_ref, rs_ref, xs_ref, w1_hbm, b1_ref,
                w2_hbm, b2_ref, out_ref, w1r, w2r, sem1, sem2):
    j = pl.program_id(0)
    cur = be_ref[j]
    slot = bs_ref[j]
    jm1 = jnp.maximum(j - 1, 0)
    run_start = jnp.logical_or(j == 0, cur != be_ref[jm1])

    @pl.when(j == 0)
    def _():
        c1, c2 = _w_copy(w1_hbm, w2_hbm, w1r, w2r, sem1, sem2,
                         be_ref[0], bs_ref[0])
        c1.start()
        c2.start()

    @pl.when(run_start)
    def _():
        c1, c2 = _w_copy(w1_hbm, w2_hbm, w1r, w2r, sem1, sem2, cur, slot)
        c1.wait()
        c2.wait()
        nxt = rs_ref[j]

        @pl.when(nxt < NBLK)
        def _():
            nj = jnp.minimum(nxt, NBLK - 1)
            c1, c2 = _w_copy(w1_hbm, w2_hbm, w1r, w2r, sem1, sem2,
                             be_ref[nj], bs_ref[nj])
            c1.start()
            c2.start()

    @pl.when(j < nt_ref[0])
    def _():
        xs = xs_ref[...].astype(jnp.bfloat16)
        h = jnp.dot(xs, w1r[slot].astype(jnp.bfloat16),
                    preferred_element_type=jnp.float32)
        h = jnp.maximum(h + b1_ref[0], 0.0)
        o = jnp.dot(h.astype(jnp.bfloat16), w2r[slot].astype(jnp.bfloat16),
                    preferred_element_type=jnp.float32)
        out_ref[...] = o + b2_ref[0]


def _combine_kernel(os_hbm, posk_hbm, wbig_hbm, res_out,
                    g0a, g1a, g2a, g0b, g1b, g2b,
                    i0a, i1a, i2a, i0b, i1b, i2b,
                    wb0, wb1, wb2,
                    s0a, s1a, s2a, s0b, s1b, s2b):
    wid = jax.lax.axis_index("s") * 2 + jax.lax.axis_index("c")
    t0 = wid * TW
    gsets = ((g0a, g1a, g2a), (g0b, g1b, g2b))
    isets = ((i0a, i1a, i2a), (i0b, i1b, i2b))
    ssets = ((s0a, s1a, s2a), (s0b, s1b, s2b))
    wbs = (wb0, wb1, wb2)
    cw = TW // 4
    NC = D_OUT // 16

    def fire(c, par):
        base = t0 + c * cw
        cps = []
        for k in range(K):
            pltpu.sync_copy(posk_hbm.at[pl.ds(k * T + base, cw)],
                            isets[par][k])
            cps.append(pltpu.async_copy(os_hbm.at[isets[par][k]],
                                        gsets[par][k], ssets[par][k]))
        return cps

    inflight = fire(0, 0)
    for c in range(4):
        par = c % 2
        base = t0 + c * cw
        for k in range(K):
            pltpu.sync_copy(wbig_hbm.at[pl.ds((k * T + base) * 16, cw * 16)],
                            wbs[k])
        for cp in inflight:
            cp.wait()
        if c + 1 < 4:
            inflight = fire(c + 1, 1 - par)
        g0, g1, g2 = gsets[par]

        def row_body(r, _):
            w0 = wb0[pl.ds(r * 16, 16)]
            w1 = wb1[pl.ds(r * 16, 16)]
            w2 = wb2[pl.ds(r * 16, 16)]
            for c2 in range(NC):
                sl = pl.ds(c2 * 16, 16)
                g0[r, sl] = g0[r, sl] * w0 + g1[r, sl] * w1 + g2[r, sl] * w2
            return 0

        jax.lax.fori_loop(0, cw, row_body, 0)
        pltpu.sync_copy(g0, res_out.at[pl.ds(base, cw), :])


def kernel(x, Wg, bg, W1, b1, W2, b2):
    bg2 = bg.reshape(1, E)
    posk, wk, meta, wbig = pl.pallas_call(
        _routing_kernel,
        out_shape=(
            jax.ShapeDtypeStruct((K, T), jnp.int32),
            jax.ShapeDtypeStruct((K, T), jnp.float32),
            jax.ShapeDtypeStruct((4, NBLK), jnp.int32),
            jax.ShapeDtypeStruct((K * T, 16), jnp.float32),
        ),
        in_specs=[
            pl.BlockSpec((T, D_IN), lambda: (0, 0)),
            pl.BlockSpec((D_IN, E), lambda: (0, 0)),
            pl.BlockSpec((1, E), lambda: (0, 0)),
        ],
        out_specs=(
            pl.BlockSpec((K, T), lambda: (0, 0)),
            pl.BlockSpec((K, T), lambda: (0, 0)),
            pl.BlockSpec((4, NBLK), lambda: (0, 0)),
            pl.BlockSpec((K * T, 16), lambda: (0, 0)),
        ),
    )(x, Wg, bg2)

    mesh = plsc.VectorSubcoreMesh(core_axis_name="c", subcore_axis_name="s")
    dispatch = functools.partial(
        pl.kernel,
        out_type=jax.ShapeDtypeStruct((R, D_IN), jnp.float32),
        mesh=mesh,
        scratch_types=[
            pltpu.VMEM((TW, D_IN), jnp.float32),
            pltpu.VMEM((TW,), jnp.int32),
            pltpu.VMEM((TW,), jnp.int32),
            pltpu.VMEM((TW,), jnp.int32),
            pltpu.SemaphoreType.DMA,
            pltpu.SemaphoreType.DMA,
            pltpu.SemaphoreType.DMA,
        ],
    )(_dispatch_kernel)
    x_sorted = dispatch(posk.reshape(NPAIR), x)

    be = meta[0]
    bs = meta[1]
    out_sorted = pl.pallas_call(
        _ffn_kernel,
        grid_spec=pltpu.PrefetchScalarGridSpec(
            num_scalar_prefetch=4,
            grid=(NBLK,),
            in_specs=[
                pl.BlockSpec((B, D_IN), lambda j, be, bs, nt, rs: (j, 0)),
                pl.BlockSpec(memory_space=pl.ANY),
                pl.BlockSpec((1, 1, D_H), lambda j, be, bs, nt, rs: (be[j], 0, 0)),
                pl.BlockSpec(memory_space=pl.ANY),
                pl.BlockSpec((1, 1, D_OUT),
                             lambda j, be, bs, nt, rs: (be[j], 0, 0)),
            ],
            out_specs=pl.BlockSpec((B, D_OUT), lambda j, be, bs, nt, rs: (j, 0)),
            scratch_shapes=[
                pltpu.VMEM((NBUF, D_IN, D_H), jnp.float32),
                pltpu.VMEM((NBUF, D_H, D_OUT), jnp.float32),
                pltpu.SemaphoreType.DMA((NBUF,)),
                pltpu.SemaphoreType.DMA((NBUF,)),
            ],
        ),
        out_shape=jax.ShapeDtypeStruct((R, D_OUT), jnp.float32),
        compiler_params=pltpu.CompilerParams(
            dimension_semantics=("arbitrary",),
        ),
    )(be, bs, meta[2, :1], meta[3], x_sorted,
      W1, b1.reshape(E, 1, D_H), W2, b2.reshape(E, 1, D_OUT))

    combine = functools.partial(
        pl.kernel,
        out_type=jax.ShapeDtypeStruct((T, D_OUT), jnp.float32),
        mesh=mesh,
        scratch_types=(
            [pltpu.VMEM((TW // 4, D_OUT), jnp.float32)] * 6
            + [pltpu.VMEM((TW // 4,), jnp.int32)] * 6
            + [pltpu.VMEM((TW * 4,), jnp.float32)] * 3
            + [pltpu.SemaphoreType.DMA] * 6
        ),
    )(_combine_kernel)
    result = combine(out_sorted, posk.reshape(NPAIR), wbig.reshape(K * T * 16))
    return result
